# Initial kernel scaffold; baseline (speedup 1.0000x reference)
import jax
import jax.numpy as jnp
from jax.experimental import pallas as pl


def kernel(x, edge_index, W0, b0, ln_w, ln_b, W1, b1):
    raise NotImplementedError("write your pallas kernel here")


# speedup vs baseline: 19.8879x; 19.8879x over previous
import functools

import jax
import jax.numpy as jnp
from jax import lax
from jax.experimental import pallas as pl
from jax.experimental.pallas import tpu as pltpu
from jax.experimental.pallas import tpu_sc as plsc

N = 10000
E = 320000
D = 128

NC, NS = 2, 16
NW = NC * NS
EPT = E // NW
CH = 125
CPT = EPT // CH

RPT_A = 632
RPT_LAST = N - RPT_A * (NS - 1)

_mesh = plsc.VectorSubcoreMesh(
    core_axis_name="c", subcore_axis_name="s", num_cores=NC, num_subcores=NS
)


@functools.partial(
    pl.kernel,
    out_type=jax.ShapeDtypeStruct((NC, N, 16), jnp.float32),
    mesh=_mesh,
    scratch_types=[
        pltpu.VMEM((CPT, CH), jnp.int32),
        pltpu.VMEM((CH, 16), jnp.float32),
        pltpu.VMEM_SHARED((N, 16), jnp.float32),
    ],
)
def _deg_kernel(src3_hbm, ones_hbm, zeros_hbm, out_hbm, srci, ones_v, acc):
    c = lax.axis_index("c")
    s = lax.axis_index("s")
    wid = c * NS + s

    @pl.when(s < NS - 1)
    def _():
        r0 = s * RPT_A
        pltpu.sync_copy(zeros_hbm.at[pl.ds(r0, RPT_A)], acc.at[pl.ds(r0, RPT_A)])

    @pl.when(s == NS - 1)
    def _():
        r0 = (NS - 1) * RPT_A
        pltpu.sync_copy(zeros_hbm.at[pl.ds(r0, RPT_LAST)],
                        acc.at[pl.ds(r0, RPT_LAST)])

    pltpu.sync_copy(ones_hbm, ones_v)
    pltpu.sync_copy(src3_hbm.at[wid], srci)
    plsc.subcore_barrier()

    def body(j, carry):
        pltpu.sync_copy(ones_v, acc.at[srci.at[j]], add=True)
        return carry

    lax.fori_loop(0, CPT, body, 0)
    plsc.subcore_barrier()

    @pl.when(s < NS - 1)
    def _():
        r0 = s * RPT_A
        pltpu.sync_copy(acc.at[pl.ds(r0, RPT_A)],
                        out_hbm.at[c].at[pl.ds(r0, RPT_A)])

    @pl.when(s == NS - 1)
    def _():
        r0 = (NS - 1) * RPT_A
        pltpu.sync_copy(acc.at[pl.ds(r0, RPT_LAST)],
                        out_hbm.at[c].at[pl.ds(r0, RPT_LAST)])


@functools.partial(
    pl.kernel,
    out_type=jax.ShapeDtypeStruct((NC, N, D), jnp.float32),
    mesh=_mesh,
    scratch_types=[
        pltpu.VMEM((CPT // 2, CH), jnp.int32),
        pltpu.VMEM((CPT // 2, CH), jnp.int32),
        pltpu.VMEM((CH, D), jnp.float32),
        pltpu.VMEM((CH, D), jnp.float32),
        pltpu.VMEM_SHARED((N, D), jnp.float32),
        pltpu.SemaphoreType.DMA,
        pltpu.SemaphoreType.DMA,
    ],
)
def _spmm_kernel(xs_hbm, dst3_hbm, src3_hbm, zeros_hbm, out_hbm,
                 dsti, srci, buf0, buf1, acc, sem0, sem1):
    c = lax.axis_index("c")
    s = lax.axis_index("s")
    wid = c * NS + s

    @pl.when(s < NS - 1)
    def _():
        r0 = s * RPT_A
        pltpu.sync_copy(zeros_hbm.at[pl.ds(r0, RPT_A)], acc.at[pl.ds(r0, RPT_A)])

    @pl.when(s == NS - 1)
    def _():
        r0 = (NS - 1) * RPT_A
        pltpu.sync_copy(zeros_hbm.at[pl.ds(r0, RPT_LAST)],
                        acc.at[pl.ds(r0, RPT_LAST)])

    plsc.subcore_barrier()

    def body(t, carry):
        j0 = 2 * t
        j1 = 2 * t + 1
        h0 = pltpu.async_copy(xs_hbm.at[dsti.at[j0]], buf0, sem0)
        h1 = pltpu.async_copy(xs_hbm.at[dsti.at[j1]], buf1, sem1)
        h0.wait()
        pltpu.sync_copy(buf0, acc.at[srci.at[j0]], add=True)
        h1.wait()
        pltpu.sync_copy(buf1, acc.at[srci.at[j1]], add=True)
        return carry

    for stage in range(2):
        half = CPT // 2
        pltpu.sync_copy(dst3_hbm.at[wid].at[pl.ds(stage * half, half)], dsti)
        pltpu.sync_copy(src3_hbm.at[wid].at[pl.ds(stage * half, half)], srci)
        lax.fori_loop(0, half // 2, body, 0)
    plsc.subcore_barrier()

    @pl.when(s < NS - 1)
    def _():
        r0 = s * RPT_A
        pltpu.sync_copy(acc.at[pl.ds(r0, RPT_A)],
                        out_hbm.at[c].at[pl.ds(r0, RPT_A)])

    @pl.when(s == NS - 1)
    def _():
        r0 = (NS - 1) * RPT_A
        pltpu.sync_copy(acc.at[pl.ds(r0, RPT_LAST)],
                        out_hbm.at[c].at[pl.ds(r0, RPT_LAST)])


RB = 2000


def _scale_body(p0, p1, x, xs, dinvb):
    deg = p0[:, 0:1] + p1[:, 0:1]
    dinv = lax.rsqrt(deg)
    xs[...] = x[...] * dinv
    dinvb[...] = jnp.broadcast_to(dinv, (RB, D))


_scale_call = pl.pallas_call(
    _scale_body,
    grid=(N // RB,),
    in_specs=[
        pl.BlockSpec((RB, 16), lambda i: (i, 0)),
        pl.BlockSpec((RB, 16), lambda i: (i, 0)),
        pl.BlockSpec((RB, D), lambda i: (i, 0)),
    ],
    out_specs=[
        pl.BlockSpec((RB, D), lambda i: (i, 0)),
        pl.BlockSpec((RB, D), lambda i: (i, 0)),
    ],
    out_shape=[
        jax.ShapeDtypeStruct((N, D), jnp.float32),
        jax.ShapeDtypeStruct((N, D), jnp.float32),
    ],
)


def _mlp_body(y0, y1, dinvb, w, b, lnw, lnb, o):
    x1 = (y0[...] + y1[...]) * dinvb[...]
    h = jnp.dot(x1, w[...], preferred_element_type=jnp.float32) + b[...]
    mu = jnp.mean(h, axis=-1, keepdims=True)
    var = jnp.mean((h - mu) ** 2, axis=-1, keepdims=True)
    hn = (h - mu) * lax.rsqrt(var + 1e-5) * lnw[...] + lnb[...]
    act = jnp.where(hn >= 0, hn, 0.01 * hn)
    o[...] = act * dinvb[...]


_mlp_call = pl.pallas_call(
    _mlp_body,
    grid=(N // RB,),
    in_specs=[
        pl.BlockSpec((RB, D), lambda i: (i, 0)),
        pl.BlockSpec((RB, D), lambda i: (i, 0)),
        pl.BlockSpec((RB, D), lambda i: (i, 0)),
        pl.BlockSpec((D, D), lambda i: (0, 0)),
        pl.BlockSpec((1, D), lambda i: (0, 0)),
        pl.BlockSpec((1, D), lambda i: (0, 0)),
        pl.BlockSpec((1, D), lambda i: (0, 0)),
    ],
    out_specs=pl.BlockSpec((RB, D), lambda i: (i, 0)),
    out_shape=jax.ShapeDtypeStruct((N, D), jnp.float32),
)


def _final_body(y0, y1, dinvb, w, b, o):
    x1 = (y0[...] + y1[...]) * dinvb[...]
    o[...] = jnp.dot(x1, w[...], preferred_element_type=jnp.float32) + b[...]


_final_call = pl.pallas_call(
    _final_body,
    grid=(N // RB,),
    in_specs=[
        pl.BlockSpec((RB, D), lambda i: (i, 0)),
        pl.BlockSpec((RB, D), lambda i: (i, 0)),
        pl.BlockSpec((RB, D), lambda i: (i, 0)),
        pl.BlockSpec((D, D), lambda i: (0, 0)),
        pl.BlockSpec((1, D), lambda i: (0, 0)),
    ],
    out_specs=pl.BlockSpec((RB, D), lambda i: (i, 0)),
    out_shape=jax.ShapeDtypeStruct((N, D), jnp.float32),
)


def kernel(x, edge_index, W0, b0, ln_w, ln_b, W1, b1):
    src3 = edge_index[0].reshape(NW, CPT, CH)
    dst3 = edge_index[1].reshape(NW, CPT, CH)
    ones16 = jnp.ones((CH, 16), jnp.float32)
    zeros16 = jnp.zeros((N, 16), jnp.float32)
    zeros128 = jnp.zeros((N, D), jnp.float32)

    degp = _deg_kernel(src3, ones16, zeros16)
    xs, dinvb = _scale_call(degp[0], degp[1], x)
    yp = _spmm_kernel(xs, dst3, src3, zeros128)
    xs2 = _mlp_call(yp[0], yp[1], dinvb, W0.T,
                    b0.reshape(1, D), ln_w.reshape(1, D), ln_b.reshape(1, D))
    yp2 = _spmm_kernel(xs2, dst3, src3, zeros128)
    out = _final_call(yp2[0], yp2[1], dinvb, W1.T, b1.reshape(1, D))
    return out


# speedup vs baseline: 20.6567x; 1.0387x over previous
import functools

import jax
import jax.numpy as jnp
from jax import lax
from jax.experimental import pallas as pl
from jax.experimental.pallas import tpu as pltpu
from jax.experimental.pallas import tpu_sc as plsc

N = 10000
E = 320000
D = 128

NC, NS = 2, 16
NW = NC * NS
EPT = E // NW
CH = 125
CPT = EPT // CH

RPT_A = 632
RPT_LAST = N - RPT_A * (NS - 1)

_mesh = plsc.VectorSubcoreMesh(
    core_axis_name="c", subcore_axis_name="s", num_cores=NC, num_subcores=NS
)


@functools.partial(
    pl.kernel,
    out_type=jax.ShapeDtypeStruct((NC, N, 16), jnp.float32),
    mesh=_mesh,
    scratch_types=[
        pltpu.VMEM((CPT, CH), jnp.int32),
        pltpu.VMEM((CH, 16), jnp.float32),
        pltpu.VMEM_SHARED((N, 16), jnp.float32),
        pltpu.SemaphoreType.DMA,
    ],
)
def _deg_kernel(src3_hbm, ones_hbm, zeros_hbm, out_hbm, srci, ones_v, acc, sem):
    c = lax.axis_index("c")
    s = lax.axis_index("s")
    wid = c * NS + s

    @pl.when(s < NS - 1)
    def _():
        r0 = s * RPT_A
        pltpu.sync_copy(zeros_hbm.at[pl.ds(r0, RPT_A)], acc.at[pl.ds(r0, RPT_A)])

    @pl.when(s == NS - 1)
    def _():
        r0 = (NS - 1) * RPT_A
        pltpu.sync_copy(zeros_hbm.at[pl.ds(r0, RPT_LAST)],
                        acc.at[pl.ds(r0, RPT_LAST)])

    pltpu.sync_copy(ones_hbm, ones_v)
    pltpu.sync_copy(src3_hbm.at[wid], srci)
    plsc.subcore_barrier()

    def body(j, carry):
        pltpu.async_copy(ones_v, acc.at[srci.at[j]], sem, add=True)
        return carry

    lax.fori_loop(0, CPT, body, 0)

    def drain(j, carry):
        pltpu.make_async_copy(ones_v, acc.at[srci.at[j]], sem).wait()
        return carry

    lax.fori_loop(0, CPT, drain, 0)
    plsc.subcore_barrier()

    @pl.when(s < NS - 1)
    def _():
        r0 = s * RPT_A
        pltpu.sync_copy(acc.at[pl.ds(r0, RPT_A)],
                        out_hbm.at[c].at[pl.ds(r0, RPT_A)])

    @pl.when(s == NS - 1)
    def _():
        r0 = (NS - 1) * RPT_A
        pltpu.sync_copy(acc.at[pl.ds(r0, RPT_LAST)],
                        out_hbm.at[c].at[pl.ds(r0, RPT_LAST)])


@functools.partial(
    pl.kernel,
    out_type=jax.ShapeDtypeStruct((NC, N, D), jnp.float32),
    mesh=_mesh,
    scratch_types=[
        pltpu.VMEM((CPT // 2, CH), jnp.int32),
        pltpu.VMEM((CPT // 2, CH), jnp.int32),
        pltpu.VMEM((CH, D), jnp.float32),
        pltpu.VMEM((CH, D), jnp.float32),
        pltpu.VMEM_SHARED((N, D), jnp.float32),
        pltpu.SemaphoreType.DMA,
        pltpu.SemaphoreType.DMA,
        pltpu.SemaphoreType.DMA,
        pltpu.SemaphoreType.DMA,
    ],
)
def _spmm_kernel(xs_hbm, dst3_hbm, src3_hbm, zeros_hbm, out_hbm,
                 dsti, srci, buf0, buf1, acc, semg0, semg1, sems0, sems1):
    c = lax.axis_index("c")
    s = lax.axis_index("s")
    wid = c * NS + s

    @pl.when(s < NS - 1)
    def _():
        r0 = s * RPT_A
        pltpu.sync_copy(zeros_hbm.at[pl.ds(r0, RPT_A)], acc.at[pl.ds(r0, RPT_A)])

    @pl.when(s == NS - 1)
    def _():
        r0 = (NS - 1) * RPT_A
        pltpu.sync_copy(zeros_hbm.at[pl.ds(r0, RPT_LAST)],
                        acc.at[pl.ds(r0, RPT_LAST)])

    plsc.subcore_barrier()

    half = CPT // 2

    def pair(u, carry):
        j0 = 2 * u
        j1 = 2 * u + 1

        @pl.when(u > 0)
        def _():
            pltpu.make_async_copy(buf0, acc.at[srci.at[j0 - 2]], sems0).wait()

        hg0 = pltpu.async_copy(xs_hbm.at[dsti.at[j0]], buf0, semg0)

        @pl.when(u > 0)
        def _():
            pltpu.make_async_copy(buf1, acc.at[srci.at[j1 - 2]], sems1).wait()

        hg1 = pltpu.async_copy(xs_hbm.at[dsti.at[j1]], buf1, semg1)
        hg0.wait()
        pltpu.async_copy(buf0, acc.at[srci.at[j0]], sems0, add=True)
        hg1.wait()
        pltpu.async_copy(buf1, acc.at[srci.at[j1]], sems1, add=True)
        return carry

    for stage in range(2):
        pltpu.sync_copy(dst3_hbm.at[wid].at[pl.ds(stage * half, half)], dsti)
        pltpu.sync_copy(src3_hbm.at[wid].at[pl.ds(stage * half, half)], srci)
        lax.fori_loop(0, half // 2, pair, 0)
        pltpu.make_async_copy(buf0, acc.at[srci.at[half - 2]], sems0).wait()
        pltpu.make_async_copy(buf1, acc.at[srci.at[half - 1]], sems1).wait()
    plsc.subcore_barrier()

    @pl.when(s < NS - 1)
    def _():
        r0 = s * RPT_A
        pltpu.sync_copy(acc.at[pl.ds(r0, RPT_A)],
                        out_hbm.at[c].at[pl.ds(r0, RPT_A)])

    @pl.when(s == NS - 1)
    def _():
        r0 = (NS - 1) * RPT_A
        pltpu.sync_copy(acc.at[pl.ds(r0, RPT_LAST)],
                        out_hbm.at[c].at[pl.ds(r0, RPT_LAST)])


RB = 2000


def _scale_body(p0, p1, x, xs, dinvb):
    deg = p0[:, 0:1] + p1[:, 0:1]
    dinv = lax.rsqrt(deg)
    xs[...] = x[...] * dinv
    dinvb[...] = jnp.broadcast_to(dinv, (RB, D))


_scale_call = pl.pallas_call(
    _scale_body,
    grid=(N // RB,),
    in_specs=[
        pl.BlockSpec((RB, 16), lambda i: (i, 0)),
        pl.BlockSpec((RB, 16), lambda i: (i, 0)),
        pl.BlockSpec((RB, D), lambda i: (i, 0)),
    ],
    out_specs=[
        pl.BlockSpec((RB, D), lambda i: (i, 0)),
        pl.BlockSpec((RB, D), lambda i: (i, 0)),
    ],
    out_shape=[
        jax.ShapeDtypeStruct((N, D), jnp.float32),
        jax.ShapeDtypeStruct((N, D), jnp.float32),
    ],
)


def _mlp_body(y0, y1, dinvb, w, b, lnw, lnb, o):
    x1 = (y0[...] + y1[...]) * dinvb[...]
    h = jnp.dot(x1, w[...], preferred_element_type=jnp.float32) + b[...]
    mu = jnp.mean(h, axis=-1, keepdims=True)
    var = jnp.mean((h - mu) ** 2, axis=-1, keepdims=True)
    hn = (h - mu) * lax.rsqrt(var + 1e-5) * lnw[...] + lnb[...]
    act = jnp.where(hn >= 0, hn, 0.01 * hn)
    o[...] = act * dinvb[...]


_mlp_call = pl.pallas_call(
    _mlp_body,
    grid=(N // RB,),
    in_specs=[
        pl.BlockSpec((RB, D), lambda i: (i, 0)),
        pl.BlockSpec((RB, D), lambda i: (i, 0)),
        pl.BlockSpec((RB, D), lambda i: (i, 0)),
        pl.BlockSpec((D, D), lambda i: (0, 0)),
        pl.BlockSpec((1, D), lambda i: (0, 0)),
        pl.BlockSpec((1, D), lambda i: (0, 0)),
        pl.BlockSpec((1, D), lambda i: (0, 0)),
    ],
    out_specs=pl.BlockSpec((RB, D), lambda i: (i, 0)),
    out_shape=jax.ShapeDtypeStruct((N, D), jnp.float32),
)


def _final_body(y0, y1, dinvb, w, b, o):
    x1 = (y0[...] + y1[...]) * dinvb[...]
    o[...] = jnp.dot(x1, w[...], preferred_element_type=jnp.float32) + b[...]


_final_call = pl.pallas_call(
    _final_body,
    grid=(N // RB,),
    in_specs=[
        pl.BlockSpec((RB, D), lambda i: (i, 0)),
        pl.BlockSpec((RB, D), lambda i: (i, 0)),
        pl.BlockSpec((RB, D), lambda i: (i, 0)),
        pl.BlockSpec((D, D), lambda i: (0, 0)),
        pl.BlockSpec((1, D), lambda i: (0, 0)),
    ],
    out_specs=pl.BlockSpec((RB, D), lambda i: (i, 0)),
    out_shape=jax.ShapeDtypeStruct((N, D), jnp.float32),
)


def kernel(x, edge_index, W0, b0, ln_w, ln_b, W1, b1):
    src3 = edge_index[0].reshape(NW, CPT, CH)
    dst3 = edge_index[1].reshape(NW, CPT, CH)
    ones16 = jnp.ones((CH, 16), jnp.float32)
    zeros16 = jnp.zeros((N, 16), jnp.float32)
    zeros128 = jnp.zeros((N, D), jnp.float32)

    degp = _deg_kernel(src3, ones16, zeros16)
    xs, dinvb = _scale_call(degp[0], degp[1], x)
    yp = _spmm_kernel(xs, dst3, src3, zeros128)
    xs2 = _mlp_call(yp[0], yp[1], dinvb, W0.T,
                    b0.reshape(1, D), ln_w.reshape(1, D), ln_b.reshape(1, D))
    yp2 = _spmm_kernel(xs2, dst3, src3, zeros128)
    out = _final_call(yp2[0], yp2[1], dinvb, W1.T, b1.reshape(1, D))
    return out


# speedup vs baseline: 20.7947x; 1.0067x over previous
import functools

import jax
import jax.numpy as jnp
from jax import lax
from jax.experimental import pallas as pl
from jax.experimental.pallas import tpu as pltpu
from jax.experimental.pallas import tpu_sc as plsc

N = 10000
E = 320000
D = 128

NC, NS = 2, 16
NW = NC * NS
EPT = E // NW
CH = 125
CPT = EPT // CH

RPT_A = 632
RPT_LAST = N - RPT_A * (NS - 1)

_mesh = plsc.VectorSubcoreMesh(
    core_axis_name="c", subcore_axis_name="s", num_cores=NC, num_subcores=NS
)


@functools.partial(
    pl.kernel,
    out_type=jax.ShapeDtypeStruct((NC, N, 16), jnp.float32),
    mesh=_mesh,
    scratch_types=[
        pltpu.VMEM((CPT, CH), jnp.int32),
        pltpu.VMEM((CH, 16), jnp.float32),
        pltpu.VMEM_SHARED((N, 16), jnp.float32),
        pltpu.SemaphoreType.DMA,
    ],
)
def _deg_kernel(src3_hbm, ones_hbm, zeros_hbm, out_hbm, srci, ones_v, acc, sem):
    c = lax.axis_index("c")
    s = lax.axis_index("s")
    wid = c * NS + s

    @pl.when(s < NS - 1)
    def _():
        r0 = s * RPT_A
        pltpu.async_copy(zeros_hbm.at[pl.ds(r0, RPT_A)],
                         acc.at[pl.ds(r0, RPT_A)], sem)

    @pl.when(s == NS - 1)
    def _():
        r0 = (NS - 1) * RPT_A
        pltpu.async_copy(zeros_hbm.at[pl.ds(r0, RPT_LAST)],
                         acc.at[pl.ds(r0, RPT_LAST)], sem)

    pltpu.sync_copy(ones_hbm, ones_v)
    pltpu.sync_copy(src3_hbm.at[wid], srci)

    @pl.when(s < NS - 1)
    def _():
        r0 = s * RPT_A
        pltpu.make_async_copy(zeros_hbm.at[pl.ds(r0, RPT_A)],
                              acc.at[pl.ds(r0, RPT_A)], sem).wait()

    @pl.when(s == NS - 1)
    def _():
        r0 = (NS - 1) * RPT_A
        pltpu.make_async_copy(zeros_hbm.at[pl.ds(r0, RPT_LAST)],
                              acc.at[pl.ds(r0, RPT_LAST)], sem).wait()

    plsc.subcore_barrier()

    def body(j, carry):
        pltpu.async_copy(ones_v, acc.at[srci.at[j]], sem, add=True)
        return carry

    lax.fori_loop(0, CPT, body, 0)

    def drain(j, carry):
        pltpu.make_async_copy(ones_v, acc.at[srci.at[j]], sem).wait()
        return carry

    lax.fori_loop(0, CPT, drain, 0)
    plsc.subcore_barrier()

    @pl.when(s < NS - 1)
    def _():
        r0 = s * RPT_A
        pltpu.sync_copy(acc.at[pl.ds(r0, RPT_A)],
                        out_hbm.at[c].at[pl.ds(r0, RPT_A)])

    @pl.when(s == NS - 1)
    def _():
        r0 = (NS - 1) * RPT_A
        pltpu.sync_copy(acc.at[pl.ds(r0, RPT_LAST)],
                        out_hbm.at[c].at[pl.ds(r0, RPT_LAST)])


@functools.partial(
    pl.kernel,
    out_type=jax.ShapeDtypeStruct((NC, N, D), jnp.float32),
    mesh=_mesh,
    scratch_types=[
        pltpu.VMEM((CPT // 2, CH), jnp.int32),
        pltpu.VMEM((CPT // 2, CH), jnp.int32),
        pltpu.VMEM((CH, D), jnp.float32),
        pltpu.VMEM((CH, D), jnp.float32),
        pltpu.VMEM_SHARED((N, D), jnp.float32),
        pltpu.SemaphoreType.DMA,
        pltpu.SemaphoreType.DMA,
        pltpu.SemaphoreType.DMA,
        pltpu.SemaphoreType.DMA,
    ],
)
def _spmm_kernel(xs_hbm, dst3_hbm, src3_hbm, zeros_hbm, out_hbm,
                 dsti, srci, buf0, buf1, acc, semg0, semg1, sems0, sems1):
    c = lax.axis_index("c")
    s = lax.axis_index("s")
    wid = c * NS + s
    half = CPT // 2

    @pl.when(s < NS - 1)
    def _():
        r0 = s * RPT_A
        pltpu.async_copy(zeros_hbm.at[pl.ds(r0, RPT_A)],
                         acc.at[pl.ds(r0, RPT_A)], semg0)

    @pl.when(s == NS - 1)
    def _():
        r0 = (NS - 1) * RPT_A
        pltpu.async_copy(zeros_hbm.at[pl.ds(r0, RPT_LAST)],
                         acc.at[pl.ds(r0, RPT_LAST)], semg0)

    pltpu.sync_copy(dst3_hbm.at[wid].at[pl.ds(0, half)], dsti)
    pltpu.sync_copy(src3_hbm.at[wid].at[pl.ds(0, half)], srci)

    @pl.when(s < NS - 1)
    def _():
        r0 = s * RPT_A
        pltpu.make_async_copy(zeros_hbm.at[pl.ds(r0, RPT_A)],
                              acc.at[pl.ds(r0, RPT_A)], semg0).wait()

    @pl.when(s == NS - 1)
    def _():
        r0 = (NS - 1) * RPT_A
        pltpu.make_async_copy(zeros_hbm.at[pl.ds(r0, RPT_LAST)],
                              acc.at[pl.ds(r0, RPT_LAST)], semg0).wait()

    plsc.subcore_barrier()

    def pair(u, carry):
        j0 = 2 * u
        j1 = 2 * u + 1

        @pl.when(u > 0)
        def _():
            pltpu.make_async_copy(buf0, acc.at[srci.at[j0 - 2]], sems0).wait()

        hg0 = pltpu.async_copy(xs_hbm.at[dsti.at[j0]], buf0, semg0)

        @pl.when(u > 0)
        def _():
            pltpu.make_async_copy(buf1, acc.at[srci.at[j1 - 2]], sems1).wait()

        hg1 = pltpu.async_copy(xs_hbm.at[dsti.at[j1]], buf1, semg1)
        hg0.wait()
        pltpu.async_copy(buf0, acc.at[srci.at[j0]], sems0, add=True)
        hg1.wait()
        pltpu.async_copy(buf1, acc.at[srci.at[j1]], sems1, add=True)
        return carry

    for stage in range(2):
        if stage > 0:
            pltpu.sync_copy(dst3_hbm.at[wid].at[pl.ds(stage * half, half)], dsti)
            pltpu.sync_copy(src3_hbm.at[wid].at[pl.ds(stage * half, half)], srci)
        lax.fori_loop(0, half // 2, pair, 0)
        pltpu.make_async_copy(buf0, acc.at[srci.at[half - 2]], sems0).wait()
        pltpu.make_async_copy(buf1, acc.at[srci.at[half - 1]], sems1).wait()
    plsc.subcore_barrier()

    @pl.when(s < NS - 1)
    def _():
        r0 = s * RPT_A
        pltpu.sync_copy(acc.at[pl.ds(r0, RPT_A)],
                        out_hbm.at[c].at[pl.ds(r0, RPT_A)])

    @pl.when(s == NS - 1)
    def _():
        r0 = (NS - 1) * RPT_A
        pltpu.sync_copy(acc.at[pl.ds(r0, RPT_LAST)],
                        out_hbm.at[c].at[pl.ds(r0, RPT_LAST)])


RB = 2000


def _scale_body(p0, p1, x, xs, dinvb):
    deg = p0[:, 0:1] + p1[:, 0:1]
    dinv = lax.rsqrt(deg)
    xs[...] = x[...] * dinv
    dinvb[...] = jnp.broadcast_to(dinv, (RB, D))


_scale_call = pl.pallas_call(
    _scale_body,
    grid=(N // RB,),
    in_specs=[
        pl.BlockSpec((RB, 16), lambda i: (i, 0)),
        pl.BlockSpec((RB, 16), lambda i: (i, 0)),
        pl.BlockSpec((RB, D), lambda i: (i, 0)),
    ],
    out_specs=[
        pl.BlockSpec((RB, D), lambda i: (i, 0)),
        pl.BlockSpec((RB, D), lambda i: (i, 0)),
    ],
    out_shape=[
        jax.ShapeDtypeStruct((N, D), jnp.float32),
        jax.ShapeDtypeStruct((N, D), jnp.float32),
    ],
)


def _mlp_body(y0, y1, dinvb, w, b, lnw, lnb, o):
    x1 = (y0[...] + y1[...]) * dinvb[...]
    h = jnp.dot(x1, w[...], preferred_element_type=jnp.float32) + b[...]
    mu = jnp.mean(h, axis=-1, keepdims=True)
    var = jnp.mean((h - mu) ** 2, axis=-1, keepdims=True)
    hn = (h - mu) * lax.rsqrt(var + 1e-5) * lnw[...] + lnb[...]
    act = jnp.where(hn >= 0, hn, 0.01 * hn)
    o[...] = act * dinvb[...]


_mlp_call = pl.pallas_call(
    _mlp_body,
    grid=(N // RB,),
    in_specs=[
        pl.BlockSpec((RB, D), lambda i: (i, 0)),
        pl.BlockSpec((RB, D), lambda i: (i, 0)),
        pl.BlockSpec((RB, D), lambda i: (i, 0)),
        pl.BlockSpec((D, D), lambda i: (0, 0)),
        pl.BlockSpec((1, D), lambda i: (0, 0)),
        pl.BlockSpec((1, D), lambda i: (0, 0)),
        pl.BlockSpec((1, D), lambda i: (0, 0)),
    ],
    out_specs=pl.BlockSpec((RB, D), lambda i: (i, 0)),
    out_shape=jax.ShapeDtypeStruct((N, D), jnp.float32),
)


def _final_body(y0, y1, dinvb, w, b, o):
    x1 = (y0[...] + y1[...]) * dinvb[...]
    o[...] = jnp.dot(x1, w[...], preferred_element_type=jnp.float32) + b[...]


_final_call = pl.pallas_call(
    _final_body,
    grid=(N // RB,),
    in_specs=[
        pl.BlockSpec((RB, D), lambda i: (i, 0)),
        pl.BlockSpec((RB, D), lambda i: (i, 0)),
        pl.BlockSpec((RB, D), lambda i: (i, 0)),
        pl.BlockSpec((D, D), lambda i: (0, 0)),
        pl.BlockSpec((1, D), lambda i: (0, 0)),
    ],
    out_specs=pl.BlockSpec((RB, D), lambda i: (i, 0)),
    out_shape=jax.ShapeDtypeStruct((N, D), jnp.float32),
)


def kernel(x, edge_index, W0, b0, ln_w, ln_b, W1, b1):
    src3 = edge_index[0].reshape(NW, CPT, CH)
    dst3 = edge_index[1].reshape(NW, CPT, CH)
    ones16 = jnp.ones((CH, 16), jnp.float32)
    zeros16 = jnp.zeros((N, 16), jnp.float32)
    zeros128 = jnp.zeros((N, D), jnp.float32)

    degp = _deg_kernel(src3, ones16, zeros16)
    xs, dinvb = _scale_call(degp[0], degp[1], x)
    yp = _spmm_kernel(xs, dst3, src3, zeros128)
    xs2 = _mlp_call(yp[0], yp[1], dinvb, W0.T,
                    b0.reshape(1, D), ln_w.reshape(1, D), ln_b.reshape(1, D))
    yp2 = _spmm_kernel(xs2, dst3, src3, zeros128)
    out = _final_call(yp2[0], yp2[1], dinvb, W1.T, b1.reshape(1, D))
    return out


# speedup vs baseline: 21.9474x; 1.0554x over previous
import functools

import jax
import jax.numpy as jnp
from jax import lax
from jax.experimental import pallas as pl
from jax.experimental.pallas import tpu as pltpu
from jax.experimental.pallas import tpu_sc as plsc

N = 10000
E = 320000
D = 128

NC, NS = 2, 16
NW = NC * NS
EPT = E // NW
CH = 125
CPT = EPT // CH

RPT_A = 632
RPT_LAST = N - RPT_A * (NS - 1)

_mesh = plsc.VectorSubcoreMesh(
    core_axis_name="c", subcore_axis_name="s", num_cores=NC, num_subcores=NS
)


@functools.partial(
    pl.kernel,
    out_type=jax.ShapeDtypeStruct((NC, N, 16), jnp.float32),
    mesh=_mesh,
    scratch_types=[
        pltpu.VMEM((CPT, CH), jnp.int32),
        pltpu.VMEM((CH, 16), jnp.float32),
        pltpu.VMEM_SHARED((N, 16), jnp.float32),
        pltpu.SemaphoreType.DMA,
    ],
)
def _deg_kernel(src3_hbm, ones_hbm, zeros_hbm, out_hbm, srci, ones_v, acc, sem):
    c = lax.axis_index("c")
    s = lax.axis_index("s")
    wid = c * NS + s

    @pl.when(s < NS - 1)
    def _():
        r0 = s * RPT_A
        pltpu.async_copy(zeros_hbm.at[pl.ds(r0, RPT_A)],
                         acc.at[pl.ds(r0, RPT_A)], sem)

    @pl.when(s == NS - 1)
    def _():
        r0 = (NS - 1) * RPT_A
        pltpu.async_copy(zeros_hbm.at[pl.ds(r0, RPT_LAST)],
                         acc.at[pl.ds(r0, RPT_LAST)], sem)

    pltpu.sync_copy(ones_hbm, ones_v)
    pltpu.sync_copy(src3_hbm.at[wid], srci)

    @pl.when(s < NS - 1)
    def _():
        r0 = s * RPT_A
        pltpu.make_async_copy(zeros_hbm.at[pl.ds(r0, RPT_A)],
                              acc.at[pl.ds(r0, RPT_A)], sem).wait()

    @pl.when(s == NS - 1)
    def _():
        r0 = (NS - 1) * RPT_A
        pltpu.make_async_copy(zeros_hbm.at[pl.ds(r0, RPT_LAST)],
                              acc.at[pl.ds(r0, RPT_LAST)], sem).wait()

    plsc.subcore_barrier()

    def body(j, carry):
        pltpu.async_copy(ones_v, acc.at[srci.at[j]], sem, add=True)
        return carry

    lax.fori_loop(0, CPT, body, 0)

    def drain(j, carry):
        pltpu.make_async_copy(ones_v, acc.at[srci.at[j]], sem).wait()
        return carry

    lax.fori_loop(0, CPT, drain, 0)
    plsc.subcore_barrier()

    @pl.when(s < NS - 1)
    def _():
        r0 = s * RPT_A
        pltpu.sync_copy(acc.at[pl.ds(r0, RPT_A)],
                        out_hbm.at[c].at[pl.ds(r0, RPT_A)])

    @pl.when(s == NS - 1)
    def _():
        r0 = (NS - 1) * RPT_A
        pltpu.sync_copy(acc.at[pl.ds(r0, RPT_LAST)],
                        out_hbm.at[c].at[pl.ds(r0, RPT_LAST)])


@functools.partial(
    pl.kernel,
    out_type=jax.ShapeDtypeStruct((NC, N, D), jnp.float32),
    mesh=_mesh,
    scratch_types=[
        pltpu.VMEM((CPT // 2, CH), jnp.int32),
        pltpu.VMEM((CPT // 2, CH), jnp.int32),
        pltpu.VMEM((CH, D), jnp.float32),
        pltpu.VMEM((CH, D), jnp.float32),
        pltpu.VMEM_SHARED((N, D), jnp.float32),
        pltpu.SemaphoreType.DMA,
        pltpu.SemaphoreType.DMA,
        pltpu.SemaphoreType.DMA,
        pltpu.SemaphoreType.DMA,
    ],
)
def _spmm_kernel(xs_hbm, dst3_hbm, src3_hbm, zeros_hbm, out_hbm,
                 dsti, srci, buf0, buf1, acc, semg0, semg1, sems0, sems1):
    c = lax.axis_index("c")
    s = lax.axis_index("s")
    wid = c * NS + s
    half = CPT // 2

    @pl.when(s < NS - 1)
    def _():
        r0 = s * RPT_A
        pltpu.async_copy(zeros_hbm.at[pl.ds(r0, RPT_A)],
                         acc.at[pl.ds(r0, RPT_A)], semg0)

    @pl.when(s == NS - 1)
    def _():
        r0 = (NS - 1) * RPT_A
        pltpu.async_copy(zeros_hbm.at[pl.ds(r0, RPT_LAST)],
                         acc.at[pl.ds(r0, RPT_LAST)], semg0)

    pltpu.sync_copy(dst3_hbm.at[wid].at[pl.ds(0, half)], dsti)
    pltpu.sync_copy(src3_hbm.at[wid].at[pl.ds(0, half)], srci)

    @pl.when(s < NS - 1)
    def _():
        r0 = s * RPT_A
        pltpu.make_async_copy(zeros_hbm.at[pl.ds(r0, RPT_A)],
                              acc.at[pl.ds(r0, RPT_A)], semg0).wait()

    @pl.when(s == NS - 1)
    def _():
        r0 = (NS - 1) * RPT_A
        pltpu.make_async_copy(zeros_hbm.at[pl.ds(r0, RPT_LAST)],
                              acc.at[pl.ds(r0, RPT_LAST)], semg0).wait()

    plsc.subcore_barrier()

    def pair(u, carry):
        j0 = 2 * u
        j1 = 2 * u + 1

        @pl.when(u > 0)
        def _():
            pltpu.make_async_copy(buf0, acc.at[srci.at[j0 - 2]], sems0).wait()

        hg0 = pltpu.async_copy(xs_hbm.at[dsti.at[j0]], buf0, semg0)

        @pl.when(u > 0)
        def _():
            pltpu.make_async_copy(buf1, acc.at[srci.at[j1 - 2]], sems1).wait()

        hg1 = pltpu.async_copy(xs_hbm.at[dsti.at[j1]], buf1, semg1)
        hg0.wait()
        pltpu.async_copy(buf0, acc.at[srci.at[j0]], sems0, add=True)
        hg1.wait()
        pltpu.async_copy(buf1, acc.at[srci.at[j1]], sems1, add=True)
        return carry

    for stage in range(2):
        if stage > 0:
            pltpu.sync_copy(dst3_hbm.at[wid].at[pl.ds(stage * half, half)], dsti)
            pltpu.sync_copy(src3_hbm.at[wid].at[pl.ds(stage * half, half)], srci)
        lax.fori_loop(0, half // 2, pair, 0)
        pltpu.make_async_copy(buf0, acc.at[srci.at[half - 2]], sems0).wait()
        pltpu.make_async_copy(buf1, acc.at[srci.at[half - 1]], sems1).wait()
    plsc.subcore_barrier()

    @pl.when(s < NS - 1)
    def _():
        r0 = s * RPT_A
        pltpu.sync_copy(acc.at[pl.ds(r0, RPT_A)],
                        out_hbm.at[c].at[pl.ds(r0, RPT_A)])

    @pl.when(s == NS - 1)
    def _():
        r0 = (NS - 1) * RPT_A
        pltpu.sync_copy(acc.at[pl.ds(r0, RPT_LAST)],
                        out_hbm.at[c].at[pl.ds(r0, RPT_LAST)])


RB = 2000


def _matT(x1, w):
    return lax.dot_general(x1, w, (((1,), (1,)), ((), ())),
                           preferred_element_type=jnp.float32)


def _scale_body(p, x, xs, dinvb):
    deg = p[0, :, 0:1] + p[1, :, 0:1]
    dinv = lax.rsqrt(deg)
    xs[...] = x[...] * dinv
    dinvb[...] = jnp.broadcast_to(dinv, (RB, D))


_scale_call = pl.pallas_call(
    _scale_body,
    grid=(N // RB,),
    in_specs=[
        pl.BlockSpec((NC, RB, 16), lambda i: (0, i, 0)),
        pl.BlockSpec((RB, D), lambda i: (i, 0)),
    ],
    out_specs=[
        pl.BlockSpec((RB, D), lambda i: (i, 0)),
        pl.BlockSpec((RB, D), lambda i: (i, 0)),
    ],
    out_shape=[
        jax.ShapeDtypeStruct((N, D), jnp.float32),
        jax.ShapeDtypeStruct((N, D), jnp.float32),
    ],
)


def _mlp_body(y, dinvb, w, b, lnw, lnb, o):
    x1 = (y[0] + y[1]) * dinvb[...]
    h = _matT(x1, w[...]) + b[...]
    mu = jnp.mean(h, axis=-1, keepdims=True)
    var = jnp.mean((h - mu) ** 2, axis=-1, keepdims=True)
    hn = (h - mu) * lax.rsqrt(var + 1e-5) * lnw[...] + lnb[...]
    act = jnp.where(hn >= 0, hn, 0.01 * hn)
    o[...] = act * dinvb[...]


_mlp_call = pl.pallas_call(
    _mlp_body,
    grid=(N // RB,),
    in_specs=[
        pl.BlockSpec((NC, RB, D), lambda i: (0, i, 0)),
        pl.BlockSpec((RB, D), lambda i: (i, 0)),
        pl.BlockSpec((D, D), lambda i: (0, 0)),
        pl.BlockSpec((1, D), lambda i: (0, 0)),
        pl.BlockSpec((1, D), lambda i: (0, 0)),
        pl.BlockSpec((1, D), lambda i: (0, 0)),
    ],
    out_specs=pl.BlockSpec((RB, D), lambda i: (i, 0)),
    out_shape=jax.ShapeDtypeStruct((N, D), jnp.float32),
)


def _final_body(y, dinvb, w, b, o):
    x1 = (y[0] + y[1]) * dinvb[...]
    o[...] = _matT(x1, w[...]) + b[...]


_final_call = pl.pallas_call(
    _final_body,
    grid=(N // RB,),
    in_specs=[
        pl.BlockSpec((NC, RB, D), lambda i: (0, i, 0)),
        pl.BlockSpec((RB, D), lambda i: (i, 0)),
        pl.BlockSpec((D, D), lambda i: (0, 0)),
        pl.BlockSpec((1, D), lambda i: (0, 0)),
    ],
    out_specs=pl.BlockSpec((RB, D), lambda i: (i, 0)),
    out_shape=jax.ShapeDtypeStruct((N, D), jnp.float32),
)


def kernel(x, edge_index, W0, b0, ln_w, ln_b, W1, b1):
    src3 = edge_index[0].reshape(NW, CPT, CH)
    dst3 = edge_index[1].reshape(NW, CPT, CH)
    ones16 = jnp.ones((CH, 16), jnp.float32)
    zeros16 = jnp.zeros((N, 16), jnp.float32)
    zeros128 = jnp.zeros((N, D), jnp.float32)

    degp = _deg_kernel(src3, ones16, zeros16)
    xs, dinvb = _scale_call(degp, x)
    yp = _spmm_kernel(xs, dst3, src3, zeros128)
    xs2 = _mlp_call(yp, dinvb, W0,
                    b0.reshape(1, D), ln_w.reshape(1, D), ln_b.reshape(1, D))
    yp2 = _spmm_kernel(xs2, dst3, src3, zeros128)
    out = _final_call(yp2, dinvb, W1, b1.reshape(1, D))
    return out
